# Initial kernel scaffold; baseline (speedup 1.0000x reference)
import jax
import jax.numpy as jnp
from jax.experimental import pallas as pl


def kernel(query_embed, operators_embedding, Wq, bq, Wo, bo):
    raise NotImplementedError("write your pallas kernel here")


# speedup vs baseline: 5.5032x; 5.5032x over previous
import jax
import jax.numpy as jnp
from jax.experimental import pallas as pl

_D = 4096
_H = 32
_L = 4
_N = 64
_THR = 0.3


def _body(q_ref, ops_ref, wq_ref, bq_ref, wo_ref, bo_ref, logp_ref, probs_ref):
    qvec = q_ref[...]
    ops = ops_ref[...]
    wq = wq_ref[...]
    wo = wo_ref[...]
    bq_all = bq_ref[...]
    bo_all = bo_ref[...]

    dn = (((1,), (1,)), ((), ()))
    qproj = jax.lax.dot_general(qvec, wq, dn, preferred_element_type=jnp.float32)
    A = jax.lax.dot_general(ops, wo[:, :_D], dn, preferred_element_type=jnp.float32)
    G = jax.lax.dot_general(ops, wo[:, _D:], dn, preferred_element_type=jnp.float32)

    row_iota = jax.lax.broadcasted_iota(jnp.int32, (_N, 1), 0)
    col_iota = jax.lax.broadcasted_iota(jnp.int32, (1, _N), 1)

    first_idx = jnp.int32(0)
    logp_rows = []
    probs_rows = []
    for l in range(_L):
        qs = qproj[:, l * _H:(l + 1) * _H] + bq_all[l:l + 1, :]
        qn = qs / jnp.maximum(jnp.sqrt(jnp.sum(qs * qs)), 1e-12)
        opsl = A[:, l * _H:(l + 1) * _H] + bo_all[l:l + 1, :]
        if l > 0:
            gmask = (row_iota == first_idx).astype(jnp.float32)
            grow = jnp.sum(G[:, l * _H:(l + 1) * _H] * gmask, axis=0, keepdims=True)
            opsl = opsl + grow
        rn = jnp.maximum(jnp.sqrt(jnp.sum(opsl * opsl, axis=1, keepdims=True)), 1e-12)
        opsn = opsl / rn
        scores = jax.lax.dot_general(qn, opsn, dn, preferred_element_type=jnp.float32)
        m = jnp.max(scores)
        e = jnp.exp(scores - m)
        s = jnp.sum(e)
        probs = e / s
        logp = scores - m - jnp.log(s)
        mask = probs > _THR
        has_any = jnp.sum(mask.astype(jnp.float32)) > 0.0
        pmax = jnp.max(probs)
        am = jnp.min(jnp.where(probs == pmax, col_iota, _N))
        sel = jnp.where(has_any, mask.astype(jnp.float32),
                        (col_iota == am).astype(jnp.float32))
        llp = jnp.sum(logp * sel)
        fm = jnp.min(jnp.where(mask, col_iota, _N))
        first_idx = jnp.where(has_any, fm, am)
        logp_rows.append(jnp.broadcast_to(llp[None, None], (1, 1)))
        probs_rows.append(probs)

    logp_ref[...] = jnp.concatenate(logp_rows, axis=0)
    probs_ref[...] = jnp.concatenate(probs_rows, axis=0)


def kernel(query_embed, operators_embedding, Wq, bq, Wo, bo):
    wq_flat = Wq.reshape(_L * _H, _D)
    wo_flat = Wo.reshape(_L * _H, 2 * _D)
    logp, probs = pl.pallas_call(
        _body,
        out_shape=(
            jax.ShapeDtypeStruct((_L, 1), jnp.float32),
            jax.ShapeDtypeStruct((_L, _N), jnp.float32),
        ),
    )(query_embed, operators_embedding, wq_flat, bq, wo_flat, bo)
    return (logp[:, 0], probs)
